# baseline (speedup 1.0000x reference)
import functools

import jax
import jax.numpy as jnp
from jax import lax
from jax.experimental import pallas as pl
from jax.experimental.pallas import tpu as pltpu
from jax.experimental.pallas import tpu_sc as plsc

B, NA, LD, AD = 1024, 8, 192, 64
D = LD + AD
E, TOPK, HEADS, FFN, HHID, BINS = 8, 2, 4, 1024, 512, 101
DH = D // HEADS
TD = NA * D

BLK = 32
ROWS = BLK * NA
NB = (TOPK * B) // BLK + E
P = NB * BLK

NC, NS = 2, 16
NW = NC * NS
RW = B // NW


def _routing_body(x_ref, gw_ref, pos0_ref, pos1_ref, be_ref, g_ref):
    logits = jnp.dot(x_ref[...], gw_ref[...],
                     preferred_element_type=jnp.float32)
    ii = lax.broadcasted_iota(jnp.int32, (B, E), 1)
    v0 = jnp.max(logits, axis=1, keepdims=True)
    i0 = jnp.min(jnp.where(logits == v0, ii, E), axis=1, keepdims=True)
    oh0 = (ii == i0)
    l2 = jnp.where(oh0, -jnp.inf, logits)
    v1 = jnp.max(l2, axis=1, keepdims=True)
    i1 = jnp.min(jnp.where(l2 == v1, ii, E), axis=1, keepdims=True)
    oh1 = (ii == i1)
    t = jnp.exp(v1 - v0)
    g0 = 1.0 / (1.0 + t)
    g_ref[...] = jnp.concatenate([g0, 1.0 - g0], axis=1)

    oh0f = oh0.astype(jnp.float32)
    oh1f = oh1.astype(jnp.float32)
    ohb = jnp.concatenate([oh0f, oh1f], axis=1).astype(jnp.bfloat16)
    ri = lax.broadcasted_iota(jnp.int32, (B, B), 0)
    ci = lax.broadcasted_iota(jnp.int32, (B, B), 1)
    tri = (ci <= ri).astype(jnp.bfloat16)
    C = jnp.dot(tri, ohb, preferred_element_type=jnp.float32)
    c_tot = C[B - 1:B, :]
    counts = c_tot[:, :E] + c_tot[:, E:]
    blocks = jnp.floor((counts + (BLK - 1)) * (1.0 / BLK))
    eye = (lax.broadcasted_iota(jnp.int32, (E, E), 0)
           == lax.broadcasted_iota(jnp.int32, (E, E), 1))
    ut = (lax.broadcasted_iota(jnp.int32, (E, E), 0)
          <= lax.broadcasted_iota(jnp.int32, (E, E), 1)).astype(jnp.float32)
    bcum = jnp.dot(blocks, ut, preferred_element_type=jnp.float32)
    bstart = bcum - blocks

    rank0 = jnp.sum(C[:, :E] * oh0f, axis=1, keepdims=True) - 1.0
    rank1 = jnp.sum((c_tot[:, :E] + C[:, E:]) * oh1f,
                    axis=1, keepdims=True) - 1.0
    s0 = jnp.sum(bstart * oh0f, axis=1, keepdims=True)
    s1 = jnp.sum(bstart * oh1f, axis=1, keepdims=True)
    pos0_ref[...] = (s0 * BLK + rank0).astype(jnp.int32)
    pos1_ref[...] = (s1 * BLK + rank1).astype(jnp.int32)

    bcum_col = lax.dot_general(eye.astype(jnp.float32), bcum,
                               (((1,), (1,)), ((), ())),
                               preferred_element_type=jnp.float32)
    bi = lax.broadcasted_iota(jnp.int32, (E, NB), 1).astype(jnp.float32)
    be = jnp.sum((bcum_col <= bi).astype(jnp.int32), axis=0, keepdims=True)
    be_ref[...] = jnp.minimum(be, E - 1)


def _routing(flat, gate_w):
    return pl.pallas_call(
        _routing_body,
        out_shape=[
            jax.ShapeDtypeStruct((B, 1), jnp.int32),
            jax.ShapeDtypeStruct((B, 1), jnp.int32),
            jax.ShapeDtypeStruct((1, NB), jnp.int32),
            jax.ShapeDtypeStruct((B, TOPK), jnp.float32),
        ],
    )(flat, gate_w)


def _sc_dispatch(tokens2d, pos0, pos1):
    mesh = plsc.VectorSubcoreMesh(core_axis_name="c", subcore_axis_name="s")

    @functools.partial(
        pl.kernel,
        mesh=mesh,
        out_type=jax.ShapeDtypeStruct((P, TD), jnp.float32),
        scratch_types=[
            pltpu.VMEM((RW,), jnp.int32),
            pltpu.VMEM((RW, TD), jnp.float32),
            pltpu.SemaphoreType.DMA,
        ],
        compiler_params=pltpu.CompilerParams(use_tc_tiling_on_sc=True),
    )
    def scatter_k(tok_hbm, p0_hbm, p1_hbm, out_hbm, idx_v, rows_v, sem):
        wid = lax.axis_index("s") * NC + lax.axis_index("c")
        base = wid * RW
        pltpu.sync_copy(tok_hbm.at[pl.ds(base, RW)], rows_v)
        pltpu.sync_copy(p0_hbm.at[pl.ds(base, RW)], idx_v)
        pltpu.async_copy(rows_v, out_hbm.at[idx_v], sem).wait()
        pltpu.sync_copy(p1_hbm.at[pl.ds(base, RW)], idx_v)
        pltpu.async_copy(rows_v, out_hbm.at[idx_v], sem).wait()

    return scatter_k(tokens2d, pos0, pos1)


def _sc_collect(y2d, pos0, pos1):
    mesh = plsc.VectorSubcoreMesh(core_axis_name="c", subcore_axis_name="s")

    @functools.partial(
        pl.kernel,
        mesh=mesh,
        out_type=jax.ShapeDtypeStruct((TOPK * B, TD), jnp.float32),
        scratch_types=[
            pltpu.VMEM((RW,), jnp.int32),
            pltpu.VMEM((RW, TD), jnp.float32),
            pltpu.SemaphoreType.DMA,
        ],
        compiler_params=pltpu.CompilerParams(use_tc_tiling_on_sc=True),
    )
    def gather_k(y_hbm, p0_hbm, p1_hbm, out_hbm, idx_v, rows_v, sem):
        wid = lax.axis_index("s") * NC + lax.axis_index("c")
        base = wid * RW
        pltpu.sync_copy(p0_hbm.at[pl.ds(base, RW)], idx_v)
        pltpu.async_copy(y_hbm.at[idx_v], rows_v, sem).wait()
        pltpu.sync_copy(rows_v, out_hbm.at[pl.ds(base, RW)])
        pltpu.sync_copy(p1_hbm.at[pl.ds(base, RW)], idx_v)
        pltpu.async_copy(y_hbm.at[idx_v], rows_v, sem).wait()
        pltpu.sync_copy(rows_v, out_hbm.at[pl.ds(B + base, RW)])

    return gather_k(y2d, pos0, pos1)


def _bdot(a, b, dn=None):
    if dn is None:
        return jnp.dot(a.astype(jnp.bfloat16), b.astype(jnp.bfloat16),
                       preferred_element_type=jnp.float32)
    return lax.dot_general(a.astype(jnp.bfloat16), b.astype(jnp.bfloat16),
                           dn, preferred_element_type=jnp.float32)


_T = (((1,), (1,)), ((), ()))


def _ln(x, g, b):
    m = jnp.mean(x, axis=-1, keepdims=True)
    d = x - m
    v = jnp.mean(d * d, axis=-1, keepdims=True)
    return d * lax.rsqrt(v + 1e-5) * g + b


def _expert_body(be_ref, x_ref, in_w_ref, in_b_ref, out_w_ref, out_b_ref,
                 ln1_g_ref, ln1_b_ref, w1_ref, b1_ref, w2_ref, b2_ref,
                 ln2_g_ref, ln2_b_ref, y_ref, o_scr):
    x = x_ref[...]
    qkv = _bdot(x, in_w_ref[0], _T) + in_b_ref[0]
    r = lax.broadcasted_iota(jnp.int32, (ROWS, ROWS), 0) // NA
    c = lax.broadcasted_iota(jnp.int32, (ROWS, ROWS), 1) // NA
    mask = r == c
    scale = 1.0 / (DH ** 0.5)
    qb = (qkv[:, :D] * scale).astype(jnp.bfloat16)
    kvb = qkv[:, D:].astype(jnp.bfloat16)
    for h in range(HEADS):
        qh = qb[:, h * DH:(h + 1) * DH]
        kh = kvb[:, h * DH:(h + 1) * DH]
        vh = kvb[:, D + h * DH:D + (h + 1) * DH]
        s = lax.dot_general(qh, kh, _T, preferred_element_type=jnp.float32)
        s = jnp.where(mask, s, -1e30)
        m = jnp.max(s, axis=1, keepdims=True)
        p = jnp.exp(s - m)
        inv = 1.0 / jnp.sum(p, axis=1, keepdims=True)
        o_scr[:, h * DH:(h + 1) * DH] = _bdot(p, vh) * inv
    o = _bdot(o_scr[...], out_w_ref[0], _T) + out_b_ref[0]
    x1 = _ln(x + o, ln1_g_ref[0], ln1_b_ref[0])
    h1 = jnp.maximum(_bdot(x1, w1_ref[0]) + b1_ref[0], 0.0)
    ff = _bdot(h1, w2_ref[0]) + b2_ref[0]
    y = _ln(x1 + ff, ln2_g_ref[0], ln2_b_ref[0])
    y_ref[...] = y


def _head_body(x0_ref, x1_ref, g_ref, w1_ref, b1_ref, w2_ref, b2_ref, o_ref):
    g = g_ref[...]
    x = x0_ref[...] * g[:, :1] + x1_ref[...] * g[:, 1:2]
    h = jnp.maximum(_bdot(x, w1_ref[...]) + b1_ref[...], 0.0)
    o_ref[...] = _bdot(h, w2_ref[...]) + b2_ref[...]


def kernel(z, actions, gate_w, in_w, in_b, out_w, out_b, ln1_g, ln1_b,
           ffn_w1, ffn_b1, ffn_w2, ffn_b2, ln2_g, ln2_b,
           head_w1, head_b1, head_w2, head_b2):
    tokens = jnp.concatenate([z, actions], axis=-1)
    flat = tokens.reshape(B, TD)

    pos0, pos1, be, gates = _routing(flat, gate_w)
    pos0 = pos0.reshape(B)
    pos1 = pos1.reshape(B)
    be = be.reshape(NB)

    sorted_x = _sc_dispatch(flat, pos0, pos1)

    wspec = lambda s1, s2: pl.BlockSpec((1, s1, s2),
                                        lambda i, be: (be[i], 0, 0))
    bspec = lambda s: pl.BlockSpec((1, 1, s), lambda i, be: (be[i], 0, 0))
    sorted_y = pl.pallas_call(
        _expert_body,
        grid_spec=pltpu.PrefetchScalarGridSpec(
            num_scalar_prefetch=1,
            grid=(NB,),
            in_specs=[
                pl.BlockSpec((ROWS, D), lambda i, be: (i, 0)),
                wspec(3 * D, D), bspec(3 * D),
                wspec(D, D), bspec(D), bspec(D), bspec(D),
                wspec(D, FFN), bspec(FFN),
                wspec(FFN, D), bspec(D), bspec(D), bspec(D),
            ],
            out_specs=pl.BlockSpec((ROWS, D), lambda i, be: (i, 0)),
            scratch_shapes=[pltpu.VMEM((ROWS, D), jnp.float32)],
        ),
        out_shape=jax.ShapeDtypeStruct((P * NA, D), jnp.float32),
    )(be, sorted_x.reshape(P * NA, D),
      in_w, in_b.reshape(E, 1, 3 * D),
      out_w, out_b.reshape(E, 1, D),
      ln1_g.reshape(E, 1, D), ln1_b.reshape(E, 1, D),
      ffn_w1, ffn_b1.reshape(E, 1, FFN),
      ffn_w2, ffn_b2.reshape(E, 1, D),
      ln2_g.reshape(E, 1, D), ln2_b.reshape(E, 1, D))

    y01 = _sc_collect(sorted_y.reshape(P, TD), pos0, pos1)

    BB = 256
    reward = pl.pallas_call(
        _head_body,
        grid=(B // BB,),
        in_specs=[
            pl.BlockSpec((BB, TD), lambda i: (i, 0)),
            pl.BlockSpec((BB, TD), lambda i: (i + B // BB, 0)),
            pl.BlockSpec((BB, TOPK), lambda i: (i, 0)),
            pl.BlockSpec((TD, HHID), lambda i: (0, 0)),
            pl.BlockSpec((1, HHID), lambda i: (0, 0)),
            pl.BlockSpec((HHID, BINS), lambda i: (0, 0)),
            pl.BlockSpec((1, BINS), lambda i: (0, 0)),
        ],
        out_specs=pl.BlockSpec((BB, BINS), lambda i: (i, 0)),
        out_shape=jax.ShapeDtypeStruct((B, BINS), jnp.float32),
    )(y01, y01, gates, head_w1, head_b1.reshape(1, HHID), head_w2,
      head_b2.reshape(1, BINS))
    return reward


# speedup vs baseline: 1.0199x; 1.0199x over previous
import functools

import jax
import jax.numpy as jnp
from jax import lax
from jax.experimental import pallas as pl
from jax.experimental.pallas import tpu as pltpu
from jax.experimental.pallas import tpu_sc as plsc

B, NA, LD, AD = 1024, 8, 192, 64
D = LD + AD
E, TOPK, HEADS, FFN, HHID, BINS = 8, 2, 4, 1024, 512, 101
DH = D // HEADS
TD = NA * D

BLK = 32
ROWS = BLK * NA
NB = (TOPK * B) // BLK + E
P = NB * BLK

NC, NS = 2, 16
NW = NC * NS
RW = B // NW


def _routing_body(x_ref, gw_ref, pos0_ref, pos1_ref, be_ref, g_ref, nb_ref):
    logits = jnp.dot(x_ref[...], gw_ref[...],
                     preferred_element_type=jnp.float32)
    ii = lax.broadcasted_iota(jnp.int32, (B, E), 1)
    v0 = jnp.max(logits, axis=1, keepdims=True)
    i0 = jnp.min(jnp.where(logits == v0, ii, E), axis=1, keepdims=True)
    oh0 = (ii == i0)
    l2 = jnp.where(oh0, -jnp.inf, logits)
    v1 = jnp.max(l2, axis=1, keepdims=True)
    i1 = jnp.min(jnp.where(l2 == v1, ii, E), axis=1, keepdims=True)
    oh1 = (ii == i1)
    t = jnp.exp(v1 - v0)
    g0 = 1.0 / (1.0 + t)
    g_ref[...] = jnp.concatenate([g0, 1.0 - g0], axis=1)

    oh0f = oh0.astype(jnp.float32)
    oh1f = oh1.astype(jnp.float32)
    ohb = jnp.concatenate([oh0f, oh1f], axis=1).astype(jnp.bfloat16)
    ri = lax.broadcasted_iota(jnp.int32, (B, B), 0)
    ci = lax.broadcasted_iota(jnp.int32, (B, B), 1)
    tri = (ci <= ri).astype(jnp.bfloat16)
    C = jnp.dot(tri, ohb, preferred_element_type=jnp.float32)
    c_tot = C[B - 1:B, :]
    counts = c_tot[:, :E] + c_tot[:, E:]
    blocks = jnp.floor((counts + (BLK - 1)) * (1.0 / BLK))
    eye = (lax.broadcasted_iota(jnp.int32, (E, E), 0)
           == lax.broadcasted_iota(jnp.int32, (E, E), 1))
    ut = (lax.broadcasted_iota(jnp.int32, (E, E), 0)
          <= lax.broadcasted_iota(jnp.int32, (E, E), 1)).astype(jnp.float32)
    bcum = jnp.dot(blocks, ut, preferred_element_type=jnp.float32)
    bstart = bcum - blocks

    rank0 = jnp.sum(C[:, :E] * oh0f, axis=1, keepdims=True) - 1.0
    rank1 = jnp.sum((c_tot[:, :E] + C[:, E:]) * oh1f,
                    axis=1, keepdims=True) - 1.0
    s0 = jnp.sum(bstart * oh0f, axis=1, keepdims=True)
    s1 = jnp.sum(bstart * oh1f, axis=1, keepdims=True)
    pos0_ref[...] = (s0 * BLK + rank0).astype(jnp.int32)
    pos1_ref[...] = (s1 * BLK + rank1).astype(jnp.int32)

    bcum_col = lax.dot_general(eye.astype(jnp.float32), bcum,
                               (((1,), (1,)), ((), ())),
                               preferred_element_type=jnp.float32)
    bi = lax.broadcasted_iota(jnp.int32, (E, NB), 1).astype(jnp.float32)
    be = jnp.sum((bcum_col <= bi).astype(jnp.int32), axis=0, keepdims=True)
    be_ref[...] = jnp.minimum(be, E - 1)
    nb_ref[...] = bcum[:, E - 1:E].astype(jnp.int32)


def _routing(flat, gate_w):
    return pl.pallas_call(
        _routing_body,
        out_shape=[
            jax.ShapeDtypeStruct((B, 1), jnp.int32),
            jax.ShapeDtypeStruct((B, 1), jnp.int32),
            jax.ShapeDtypeStruct((1, NB), jnp.int32),
            jax.ShapeDtypeStruct((B, TOPK), jnp.float32),
            jax.ShapeDtypeStruct((1, 1), jnp.int32),
        ],
    )(flat, gate_w)


def _sc_dispatch(tokens2d, pos0, pos1):
    mesh = plsc.VectorSubcoreMesh(core_axis_name="c", subcore_axis_name="s")

    @functools.partial(
        pl.kernel,
        mesh=mesh,
        out_type=jax.ShapeDtypeStruct((P, TD), jnp.float32),
        scratch_types=[
            pltpu.VMEM((RW,), jnp.int32),
            pltpu.VMEM((RW, TD), jnp.float32),
            pltpu.SemaphoreType.DMA,
        ],
    )
    def scatter_k(tok_hbm, p0_hbm, p1_hbm, out_hbm, idx_v, rows_v, sem):
        wid = lax.axis_index("s") * NC + lax.axis_index("c")
        base = wid * RW
        pltpu.sync_copy(tok_hbm.at[pl.ds(base, RW)], rows_v)
        pltpu.sync_copy(p0_hbm.at[pl.ds(base, RW)], idx_v)
        pltpu.async_copy(rows_v, out_hbm.at[idx_v], sem).wait()
        pltpu.sync_copy(p1_hbm.at[pl.ds(base, RW)], idx_v)
        pltpu.async_copy(rows_v, out_hbm.at[idx_v], sem).wait()

    return scatter_k(tokens2d, pos0, pos1)


def _sc_collect(y2d, pos0, pos1):
    mesh = plsc.VectorSubcoreMesh(core_axis_name="c", subcore_axis_name="s")

    @functools.partial(
        pl.kernel,
        mesh=mesh,
        out_type=jax.ShapeDtypeStruct((TOPK * B, TD), jnp.float32),
        scratch_types=[
            pltpu.VMEM((RW,), jnp.int32),
            pltpu.VMEM((RW, TD), jnp.float32),
            pltpu.SemaphoreType.DMA,
        ],
    )
    def gather_k(y_hbm, p0_hbm, p1_hbm, out_hbm, idx_v, rows_v, sem):
        wid = lax.axis_index("s") * NC + lax.axis_index("c")
        base = wid * RW
        pltpu.sync_copy(p0_hbm.at[pl.ds(base, RW)], idx_v)
        pltpu.async_copy(y_hbm.at[idx_v], rows_v, sem).wait()
        pltpu.sync_copy(rows_v, out_hbm.at[pl.ds(base, RW)])
        pltpu.sync_copy(p1_hbm.at[pl.ds(base, RW)], idx_v)
        pltpu.async_copy(y_hbm.at[idx_v], rows_v, sem).wait()
        pltpu.sync_copy(rows_v, out_hbm.at[pl.ds(B + base, RW)])

    return gather_k(y2d, pos0, pos1)


def _bdot(a, b, dn=None):
    if dn is None:
        return jnp.dot(a.astype(jnp.bfloat16), b.astype(jnp.bfloat16),
                       preferred_element_type=jnp.float32)
    return lax.dot_general(a.astype(jnp.bfloat16), b.astype(jnp.bfloat16),
                           dn, preferred_element_type=jnp.float32)


_T = (((1,), (1,)), ((), ()))


def _ln(x, g, b):
    m = jnp.mean(x, axis=-1, keepdims=True)
    d = x - m
    v = jnp.mean(d * d, axis=-1, keepdims=True)
    return d * lax.rsqrt(v + 1e-5) * g + b


def _expert_body(be_ref, nb_ref, x_ref, in_w_ref, in_b_ref, out_w_ref,
                 out_b_ref, ln1_g_ref, ln1_b_ref, w1_ref, b1_ref, w2_ref,
                 b2_ref, ln2_g_ref, ln2_b_ref, y_ref, o_scr):
    @pl.when(pl.program_id(0) < nb_ref[0])
    def _run():
        _expert_block(x_ref, in_w_ref, in_b_ref, out_w_ref, out_b_ref,
                      ln1_g_ref, ln1_b_ref, w1_ref, b1_ref, w2_ref, b2_ref,
                      ln2_g_ref, ln2_b_ref, y_ref, o_scr)


def _expert_block(x_ref, in_w_ref, in_b_ref, out_w_ref, out_b_ref,
                  ln1_g_ref, ln1_b_ref, w1_ref, b1_ref, w2_ref, b2_ref,
                  ln2_g_ref, ln2_b_ref, y_ref, o_scr):
    x = x_ref[...]
    qkv = _bdot(x, in_w_ref[0], _T) + in_b_ref[0]
    r = lax.broadcasted_iota(jnp.int32, (ROWS, ROWS), 0) // NA
    c = lax.broadcasted_iota(jnp.int32, (ROWS, ROWS), 1) // NA
    mask = r == c
    scale = 1.0 / (DH ** 0.5)
    qb = (qkv[:, :D] * scale).astype(jnp.bfloat16)
    kvb = qkv[:, D:].astype(jnp.bfloat16)
    for h in range(HEADS):
        qh = qb[:, h * DH:(h + 1) * DH]
        kh = kvb[:, h * DH:(h + 1) * DH]
        vh = kvb[:, D + h * DH:D + (h + 1) * DH]
        s = lax.dot_general(qh, kh, _T, preferred_element_type=jnp.float32)
        s = jnp.where(mask, s, -1e30)
        m = jnp.max(s, axis=1, keepdims=True)
        p = jnp.exp(s - m)
        inv = 1.0 / jnp.sum(p, axis=1, keepdims=True)
        o_scr[:, h * DH:(h + 1) * DH] = _bdot(p, vh) * inv
    o = _bdot(o_scr[...], out_w_ref[0], _T) + out_b_ref[0]
    x1 = _ln(x + o, ln1_g_ref[0], ln1_b_ref[0])
    h1 = jnp.maximum(_bdot(x1, w1_ref[0]) + b1_ref[0], 0.0)
    ff = _bdot(h1, w2_ref[0]) + b2_ref[0]
    y = _ln(x1 + ff, ln2_g_ref[0], ln2_b_ref[0])
    y_ref[...] = y


def _head_body(x0_ref, x1_ref, g_ref, w1_ref, b1_ref, w2_ref, b2_ref, o_ref):
    g = g_ref[...]
    x = x0_ref[...] * g[:, :1] + x1_ref[...] * g[:, 1:2]
    h = jnp.maximum(_bdot(x, w1_ref[...]) + b1_ref[...], 0.0)
    o_ref[...] = _bdot(h, w2_ref[...]) + b2_ref[...]


def kernel(z, actions, gate_w, in_w, in_b, out_w, out_b, ln1_g, ln1_b,
           ffn_w1, ffn_b1, ffn_w2, ffn_b2, ln2_g, ln2_b,
           head_w1, head_b1, head_w2, head_b2):
    tokens = jnp.concatenate([z, actions], axis=-1)
    flat = tokens.reshape(B, TD)

    pos0, pos1, be, gates, nb = _routing(flat, gate_w)
    pos0 = pos0.reshape(B)
    pos1 = pos1.reshape(B)
    be = be.reshape(NB)
    nb = nb.reshape(1)

    sorted_x = _sc_dispatch(flat, pos0, pos1)

    wspec = lambda s1, s2: pl.BlockSpec((1, s1, s2),
                                        lambda i, be, nb: (be[i], 0, 0))
    bspec = lambda s: pl.BlockSpec((1, 1, s), lambda i, be, nb: (be[i], 0, 0))
    sorted_y = pl.pallas_call(
        _expert_body,
        grid_spec=pltpu.PrefetchScalarGridSpec(
            num_scalar_prefetch=2,
            grid=(NB,),
            in_specs=[
                pl.BlockSpec((ROWS, D), lambda i, be, nb: (i, 0)),
                wspec(3 * D, D), bspec(3 * D),
                wspec(D, D), bspec(D), bspec(D), bspec(D),
                wspec(D, FFN), bspec(FFN),
                wspec(FFN, D), bspec(D), bspec(D), bspec(D),
            ],
            out_specs=pl.BlockSpec((ROWS, D), lambda i, be, nb: (i, 0)),
            scratch_shapes=[pltpu.VMEM((ROWS, D), jnp.float32)],
        ),
        out_shape=jax.ShapeDtypeStruct((P * NA, D), jnp.float32),
    )(be, nb, sorted_x.reshape(P * NA, D),
      in_w, in_b.reshape(E, 1, 3 * D),
      out_w, out_b.reshape(E, 1, D),
      ln1_g.reshape(E, 1, D), ln1_b.reshape(E, 1, D),
      ffn_w1, ffn_b1.reshape(E, 1, FFN),
      ffn_w2, ffn_b2.reshape(E, 1, D),
      ln2_g.reshape(E, 1, D), ln2_b.reshape(E, 1, D))

    y01 = _sc_collect(sorted_y.reshape(P, TD), pos0, pos1)

    BB = 256
    reward = pl.pallas_call(
        _head_body,
        grid=(B // BB,),
        in_specs=[
            pl.BlockSpec((BB, TD), lambda i: (i, 0)),
            pl.BlockSpec((BB, TD), lambda i: (i + B // BB, 0)),
            pl.BlockSpec((BB, TOPK), lambda i: (i, 0)),
            pl.BlockSpec((TD, HHID), lambda i: (0, 0)),
            pl.BlockSpec((1, HHID), lambda i: (0, 0)),
            pl.BlockSpec((HHID, BINS), lambda i: (0, 0)),
            pl.BlockSpec((1, BINS), lambda i: (0, 0)),
        ],
        out_specs=pl.BlockSpec((BB, BINS), lambda i: (i, 0)),
        out_shape=jax.ShapeDtypeStruct((B, BINS), jnp.float32),
    )(y01, y01, gates, head_w1, head_b1.reshape(1, HHID), head_w2,
      head_b2.reshape(1, BINS))
    return reward
